# baseline (speedup 1.0000x reference)
import jax
import jax.numpy as jnp
from jax import lax
from jax.experimental import pallas as pl
from jax.experimental.pallas import tpu as pltpu
from jax.experimental.pallas import tpu_sc as plsc

_NCODES = 16384
_BATCH = 4096
_D = 512
_L = 16
_NT = 16
_BPT = _BATCH // _NT
_FCH = _BPT // _L
_NCHUNKS = _BATCH // _L

_sc_params = pltpu.CompilerParams(needs_layout_passes=False)
_sc_mesh = plsc.VectorSubcoreMesh(
    core_axis_name="c", subcore_axis_name="s",
    num_cores=1, num_subcores=_NT)


def _copy_body(x_ref, o_ref):
    o_ref[...] = x_ref[...]


_tc_copy = pl.pallas_call(
    _copy_body,
    out_shape=jax.ShapeDtypeStruct((_BATCH, _D), jnp.float32),
    grid=(8,),
    in_specs=[pl.BlockSpec((_BATCH // 8, _D), lambda i: (i, 0))],
    out_specs=pl.BlockSpec((_BATCH // 8, _D), lambda i: (i, 0)),
)


def _sc_stages(idx_ref, data_ref, out_ref, fsrc_ref, fdst_ref, cnt_ref):
    @pl.core_map(
        _sc_mesh,
        compiler_params=_sc_params,
        scratch_shapes=[
            pltpu.VMEM((_BATCH,), jnp.int32),
            pltpu.VMEM((_NCODES,), jnp.int32),
            pltpu.VMEM((_FCH, _L), jnp.int32),
            pltpu.VMEM((_FCH, _L), jnp.int32),
            pltpu.VMEM((_L,), jnp.int32),
        ],
    )
    def _build(idx_v, table_v, fsrc_v, fdst_v, cnt_v):
        tid = lax.axis_index("s")
        base = tid * _BPT
        pltpu.sync_copy(idx_ref, idx_v)

        lane = lax.iota(jnp.int32, _L)
        nxt_lane = (lane + 1) & (_L - 1)
        last_lane = lane == (_L - 1)

        def chunk_step(c, carry):
            chunk = idx_v[pl.ds(c * _L, _L)]
            comp = chunk * _L + lane
            sk, _ = plsc.sort_key_val(comp, comp)
            nxt = jnp.take(sk, nxt_lane, mode="wrap")
            code = sk >> 4
            is_last = jnp.logical_or(code != (nxt >> 4), last_lane)
            pos = (sk & (_L - 1)) + c * _L
            plsc.store_scatter(table_v, [code], pos, mask=is_last)
            return carry

        lax.fori_loop(0, _NCHUNKS, chunk_step, 0, unroll=8)

        my0 = idx_v[pl.ds(base, _L)]
        s0 = plsc.load_gather(table_v, [my0])
        zero = jnp.zeros((_L,), jnp.int32)
        pad_src = jnp.take(s0, zero, mode="wrap")
        pad_dst = zero + base
        for j in range(_FCH):
            fsrc_v[j, :] = pad_src
            fdst_v[j, :] = pad_dst

        n = jnp.int32(0)
        for b in range(_FCH):
            my = idx_v[pl.ds(base + b * _L, _L)]
            s = plsc.load_gather(table_v, [my])
            rows = base + b * _L + lane
            m = s != rows
            mi = m.astype(jnp.int32)
            posn = n + jnp.cumsum(mi) - 1
            plsc.store_scatter(fsrc_v, [posn >> 4, posn & (_L - 1)], s,
                               mask=m)
            plsc.store_scatter(fdst_v, [posn >> 4, posn & (_L - 1)], rows,
                               mask=m)
            n = n + jnp.sum(mi)

        cnt_v[...] = zero + n
        pltpu.sync_copy(fsrc_v, fsrc_ref.at[tid])
        pltpu.sync_copy(fdst_v, fdst_ref.at[tid])
        pltpu.sync_copy(cnt_v, cnt_ref.at[tid])

    @pl.core_map(
        _sc_mesh,
        compiler_params=_sc_params,
        scratch_shapes=[
            pltpu.VMEM((_FCH, _L), jnp.int32),
            pltpu.VMEM((_FCH, _L), jnp.int32),
            pltpu.VMEM((_L,), jnp.int32),
            pltpu.VMEM((_L, _D), jnp.float32),
            pltpu.SemaphoreType.DMA,
            pltpu.SemaphoreType.DMA,
        ],
    )
    def _patch(fsrc_v, fdst_v, cnt_v, fbuf, gsem, wsem):
        tid = lax.axis_index("s")
        pltpu.sync_copy(fsrc_ref.at[tid], fsrc_v)
        pltpu.sync_copy(fdst_ref.at[tid], fdst_v)
        pltpu.sync_copy(cnt_ref.at[tid], cnt_v)
        n = jnp.max(cnt_v[...])
        for j in range(_FCH):
            @pl.when(j * _L < n)
            def _step():
                pltpu.async_copy(
                    data_ref.at[fsrc_v.at[j]], fbuf, gsem).wait()
                pltpu.async_copy(
                    fbuf, out_ref.at[fdst_v.at[j]], wsem).wait()


@jax.jit
def kernel(seq_idxs, set_data, cache):
    del cache
    out0 = _tc_copy(set_data)
    fsrc0 = jnp.zeros((_NT, _FCH, _L), jnp.int32)
    fdst0 = jnp.zeros((_NT, _FCH, _L), jnp.int32)
    cnt0 = jnp.zeros((_NT, _L), jnp.int32)

    def stateful(refs):
        _sc_stages(*refs)

    _, _, out, _, _, _ = pl.run_state(stateful)(
        (seq_idxs.astype(jnp.int32), set_data, out0, fsrc0, fdst0, cnt0))
    return out


# speedup vs baseline: 1.1924x; 1.1924x over previous
import functools

import jax
import jax.numpy as jnp
from jax import lax
from jax.experimental import pallas as pl
from jax.experimental.pallas import tpu as pltpu
from jax.experimental.pallas import tpu_sc as plsc

_NCODES = 16384
_BATCH = 4096
_D = 512
_L = 16
_NT = 16
_BPT = _BATCH // _NT
_RC = 64
_NCH = _BPT // _RC
_NCHUNKS = _BATCH // _L


def _body(idx_hbm, data_hbm, out_hbm, idx_v, table_v, src_v, buf0, buf1,
          gs0, gs1, ws0, ws1):
    tid = lax.axis_index("s")
    base = tid * _BPT

    pltpu.sync_copy(idx_hbm, idx_v)

    lane = lax.iota(jnp.int32, _L)
    nxt_lane = (lane + 1) & (_L - 1)
    last_lane = lane == (_L - 1)

    def chunk_step(c, carry):
        chunk = idx_v[pl.ds(c * _L, _L)]
        comp = chunk * _L + lane
        sk, _ = plsc.sort_key_val(comp, comp)
        nxt = jnp.take(sk, nxt_lane, mode="wrap")
        code = sk >> 4
        is_last = jnp.logical_or(code != (nxt >> 4), last_lane)
        pos = (sk & (_L - 1)) + c * _L
        plsc.store_scatter(table_v, [code], pos, mask=is_last)
        return carry

    lax.fori_loop(0, _NCHUNKS, chunk_step, 0, unroll=8)

    for b in range(_BPT // _L):
        my = idx_v[pl.ds(base + b * _L, _L)]
        src_v[pl.ds(b * _L, _L)] = plsc.load_gather(table_v, [my])

    bufs = (buf0, buf1)
    gsems = (gs0, gs1)
    wsems = (ws0, ws1)

    def _gather(k):
        return pltpu.async_copy(
            data_hbm.at[src_v.at[pl.ds(k * _RC, _RC)]],
            bufs[k % 2], gsems[k % 2])

    def _write(k):
        return pltpu.async_copy(
            bufs[k % 2], out_hbm.at[pl.ds(base + k * _RC, _RC)],
            wsems[k % 2])

    gets = [_gather(0), _gather(1)]
    puts = []
    for k in range(_NCH):
        gets[k].wait()
        puts.append(_write(k))
        if k + 2 < _NCH:
            puts[k].wait()
            gets.append(_gather(k + 2))
    for k in (_NCH - 2, _NCH - 1):
        puts[k].wait()


_cache_lookup = functools.partial(
    pl.kernel,
    out_type=jax.ShapeDtypeStruct((_BATCH, _D), jnp.float32),
    mesh=plsc.VectorSubcoreMesh(
        core_axis_name="c", subcore_axis_name="s",
        num_cores=1, num_subcores=_NT),
    scratch_types=[
        pltpu.VMEM((_BATCH,), jnp.int32),
        pltpu.VMEM((_NCODES,), jnp.int32),
        pltpu.VMEM((_BPT,), jnp.int32),
        pltpu.VMEM((_RC, _D), jnp.float32),
        pltpu.VMEM((_RC, _D), jnp.float32),
        pltpu.SemaphoreType.DMA,
        pltpu.SemaphoreType.DMA,
        pltpu.SemaphoreType.DMA,
        pltpu.SemaphoreType.DMA,
    ],
    compiler_params=pltpu.CompilerParams(needs_layout_passes=False),
)(_body)


@jax.jit
def kernel(seq_idxs, set_data, cache):
    del cache
    return _cache_lookup(seq_idxs.astype(jnp.int32), set_data)


# speedup vs baseline: 1.2797x; 1.0732x over previous
import functools

import jax
import jax.numpy as jnp
from jax import lax
from jax.experimental import pallas as pl
from jax.experimental.pallas import tpu as pltpu
from jax.experimental.pallas import tpu_sc as plsc

_NCODES = 16384
_BATCH = 4096
_D = 512
_L = 16
_NC = 2
_NS = 16
_NW = _NC * _NS
_BPW = _BATCH // _NW
_NCHUNKS = _BATCH // _L


def _body(idx_hbm, data_hbm, out_hbm, idx_v, table_v, src_v, rows_v, sem):
    wid = lax.axis_index("s") * _NC + lax.axis_index("c")

    pltpu.sync_copy(idx_hbm, idx_v)

    lane = lax.iota(jnp.int32, _L)
    nxt_lane = (lane + 1) & (_L - 1)
    last_lane = lane == (_L - 1)

    def chunk_step(c, carry):
        chunk = idx_v[pl.ds(c * _L, _L)]
        comp = chunk * _L + lane
        sk, _ = plsc.sort_key_val(comp, comp)
        nxt = jnp.take(sk, nxt_lane, mode="wrap")
        code = sk >> 4
        is_last = jnp.logical_or(code != (nxt >> 4), last_lane)
        pos = (sk & (_L - 1)) + c * _L
        plsc.store_scatter(table_v, [code], pos, mask=is_last)
        return carry

    lax.fori_loop(0, _NCHUNKS, chunk_step, 0, unroll=8)

    base = wid * _BPW
    for b in range(_BPW // _L):
        my = idx_v[pl.ds(base + b * _L, _L)]
        src_v[pl.ds(b * _L, _L)] = plsc.load_gather(table_v, [my])

    pltpu.async_copy(data_hbm.at[src_v], rows_v, sem).wait()
    pltpu.sync_copy(rows_v, out_hbm.at[pl.ds(base, _BPW)])


_cache_lookup = functools.partial(
    pl.kernel,
    out_type=jax.ShapeDtypeStruct((_BATCH, _D), jnp.float32),
    mesh=plsc.VectorSubcoreMesh(
        core_axis_name="c", subcore_axis_name="s",
        num_cores=_NC, num_subcores=_NS),
    scratch_types=[
        pltpu.VMEM((_BATCH,), jnp.int32),
        pltpu.VMEM((_NCODES,), jnp.int32),
        pltpu.VMEM((_BPW,), jnp.int32),
        pltpu.VMEM((_BPW, _D), jnp.float32),
        pltpu.SemaphoreType.DMA,
    ],
    compiler_params=pltpu.CompilerParams(needs_layout_passes=False, use_tc_tiling_on_sc=True),
)(_body)


@jax.jit
def kernel(seq_idxs, set_data, cache):
    del cache
    return _cache_lookup(seq_idxs.astype(jnp.int32), set_data)
